# initial kernel scaffold (unmeasured)
import jax
import jax.numpy as jnp
from jax import lax
from jax.experimental import pallas as pl
from jax.experimental.pallas import tpu as pltpu

N_DEV = 4
M_BLK = 1024
K_BLK = 1024
K = 4096
N = 8192
NT = 1024
N_TILES = N // NT


def _a2a_body(x_ref, a_ref, send_buf, send_sems, recv_sems):
    my = lax.axis_index("i")

    barrier = pltpu.get_barrier_semaphore()
    for d in range(1, N_DEV):
        peer = (my + d) % N_DEV
        pl.semaphore_signal(
            barrier, inc=1, device_id=(peer,),
            device_id_type=pl.DeviceIdType.MESH,
        )
    pl.semaphore_wait(barrier, N_DEV - 1)

    for j in range(N_DEV):
        send_buf[j] = x_ref[pl.ds(j * M_BLK, M_BLK), :].astype(jnp.bfloat16)
    a_ref[my] = send_buf[my]

    rdmas = []
    for d in range(1, N_DEV):
        peer = (my + d) % N_DEV
        rdma = pltpu.make_async_remote_copy(
            src_ref=send_buf.at[peer],
            dst_ref=a_ref.at[my],
            send_sem=send_sems.at[d - 1],
            recv_sem=recv_sems.at[d - 1],
            device_id=(peer,),
            device_id_type=pl.DeviceIdType.MESH,
        )
        rdma.start()
        rdmas.append(rdma)
    for rdma in rdmas:
        rdma.wait()


def _gemm_body(a_ref, w_ref, y_ref, amax_ref):
    n = pl.program_id(0)
    acc = jnp.zeros((M_BLK, NT), jnp.float32)
    for j in range(N_DEV):
        acc += jnp.dot(
            a_ref[j],
            w_ref[pl.ds(j * K_BLK, K_BLK), :].astype(jnp.bfloat16),
            preferred_element_type=jnp.float32,
        )
    y = jnp.maximum(acc, 0.0)
    y_ref[...] = y
    m = jnp.max(y)

    @pl.when(n == 0)
    def _():
        amax_ref[...] = jnp.full((8, 128), m, jnp.float32)

    @pl.when(n > 0)
    def _():
        amax_ref[...] = jnp.maximum(amax_ref[...], m)


def _quant_body(y_ref, amax_ref, out_ref, gather, scale_ref,
                send_sems, recv_sems):
    t = pl.program_id(0)

    @pl.when(t == 0)
    def _():
        my = lax.axis_index("i")
        barrier = pltpu.get_barrier_semaphore()
        for d in range(1, N_DEV):
            peer = (my + d) % N_DEV
            pl.semaphore_signal(
                barrier, inc=1, device_id=(peer,),
                device_id_type=pl.DeviceIdType.MESH,
            )
        pl.semaphore_wait(barrier, N_DEV - 1)

        gather[N_DEV - 1] = amax_ref[...]
        rdmas = []
        for d in range(1, N_DEV):
            peer = (my + d) % N_DEV
            rdma = pltpu.make_async_remote_copy(
                src_ref=amax_ref,
                dst_ref=gather.at[d - 1],
                send_sem=send_sems.at[d - 1],
                recv_sem=recv_sems.at[d - 1],
                device_id=(peer,),
                device_id_type=pl.DeviceIdType.MESH,
            )
            rdma.start()
            rdmas.append(rdma)
        for rdma in rdmas:
            rdma.wait()
        scale_ref[0, 0] = jnp.max(gather[...]) / 448.0

    s = scale_ref[0, 0]
    q = (y_ref[...] / s).astype(jnp.float8_e4m3fn)
    out_ref[...] = q.astype(jnp.float32) * s


def kernel(x, w_mat):
    a = pl.pallas_call(
        _a2a_body,
        out_shape=jax.ShapeDtypeStruct((N_DEV, M_BLK, K_BLK), jnp.bfloat16),
        in_specs=[pl.BlockSpec(memory_space=pltpu.VMEM)],
        out_specs=pl.BlockSpec(memory_space=pltpu.VMEM),
        scratch_shapes=[
            pltpu.VMEM((N_DEV, M_BLK, K_BLK), jnp.bfloat16),
            pltpu.SemaphoreType.DMA((N_DEV - 1,)),
            pltpu.SemaphoreType.DMA((N_DEV - 1,)),
        ],
        compiler_params=pltpu.CompilerParams(collective_id=0),
    )(x)

    y, amax = pl.pallas_call(
        _gemm_body,
        grid=(N_TILES,),
        out_shape=[
            jax.ShapeDtypeStruct((M_BLK, N), jnp.float32),
            jax.ShapeDtypeStruct((8, 128), jnp.float32),
        ],
        in_specs=[
            pl.BlockSpec((N_DEV, M_BLK, K_BLK), lambda n: (0, 0, 0)),
            pl.BlockSpec((K, NT), lambda n: (0, n)),
        ],
        out_specs=[
            pl.BlockSpec((M_BLK, NT), lambda n: (0, n)),
            pl.BlockSpec((8, 128), lambda n: (0, 0)),
        ],
        compiler_params=pltpu.CompilerParams(
            dimension_semantics=("arbitrary",),
        ),
    )(a, w_mat)

    out = pl.pallas_call(
        _quant_body,
        grid=(N_TILES,),
        out_shape=jax.ShapeDtypeStruct((M_BLK, N), jnp.float32),
        in_specs=[
            pl.BlockSpec((M_BLK, NT), lambda t: (0, t)),
            pl.BlockSpec((8, 128), lambda t: (0, 0)),
        ],
        out_specs=pl.BlockSpec((M_BLK, NT), lambda t: (0, t)),
        scratch_shapes=[
            pltpu.VMEM((N_DEV, 8, 128), jnp.float32),
            pltpu.SMEM((1, 1), jnp.float32),
            pltpu.SemaphoreType.DMA((N_DEV - 1,)),
            pltpu.SemaphoreType.DMA((N_DEV - 1,)),
        ],
        compiler_params=pltpu.CompilerParams(
            dimension_semantics=("arbitrary",),
            collective_id=1,
        ),
    )(y, amax)
    return out


# baseline (device time: 198146 ns/iter reference)
import jax
import jax.numpy as jnp
from jax import lax
from jax.experimental import pallas as pl
from jax.experimental.pallas import tpu as pltpu

N_DEV = 4
M_BLK = 1024
K_BLK = 1024
K = 4096
N = 8192
NT = 1024
N_TILES = N // NT


def _a2a_body(x_ref, a_ref, send_buf, send_sems, recv_sems):
    my = lax.axis_index("i")

    barrier = pltpu.get_barrier_semaphore()
    for d in range(1, N_DEV):
        peer = (my + d) % N_DEV
        pl.semaphore_signal(
            barrier, inc=1, device_id=(peer,),
            device_id_type=pl.DeviceIdType.MESH,
        )
    pl.semaphore_wait(barrier, N_DEV - 1)

    for j in range(N_DEV):
        send_buf[j] = x_ref[pl.ds(j * M_BLK, M_BLK), :].astype(jnp.bfloat16)
    a_ref[my] = send_buf[my]

    rdmas = []
    for d in range(1, N_DEV):
        peer = (my + d) % N_DEV
        rdma = pltpu.make_async_remote_copy(
            src_ref=send_buf.at[peer],
            dst_ref=a_ref.at[my],
            send_sem=send_sems.at[d - 1],
            recv_sem=recv_sems.at[d - 1],
            device_id=(peer,),
            device_id_type=pl.DeviceIdType.MESH,
        )
        rdma.start()
        rdmas.append(rdma)
    for rdma in rdmas:
        rdma.wait()


def _gemm_body(a_ref, w_ref, y_ref, amax_ref):
    n = pl.program_id(0)
    acc = jnp.zeros((M_BLK, NT), jnp.float32)
    for j in range(N_DEV):
        acc += jnp.dot(
            a_ref[j],
            w_ref[pl.ds(j * K_BLK, K_BLK), :].astype(jnp.bfloat16),
            preferred_element_type=jnp.float32,
        )
    y = jnp.maximum(acc, 0.0)
    y_ref[...] = y
    m = jnp.max(y)

    @pl.when(n == 0)
    def _():
        amax_ref[...] = jnp.full((8, 128), m, jnp.float32)

    @pl.when(n > 0)
    def _():
        amax_ref[...] = jnp.maximum(amax_ref[...], m)


def _quant_body(y_ref, amax_ref, out_ref, gather, scale_ref,
                send_sems, recv_sems):
    t = pl.program_id(0)

    @pl.when(t == 0)
    def _():
        my = lax.axis_index("i")
        barrier = pltpu.get_barrier_semaphore()
        for d in range(1, N_DEV):
            peer = (my + d) % N_DEV
            pl.semaphore_signal(
                barrier, inc=1, device_id=(peer,),
                device_id_type=pl.DeviceIdType.MESH,
            )
        pl.semaphore_wait(barrier, N_DEV - 1)

        gather[N_DEV - 1] = amax_ref[...]
        rdmas = []
        for d in range(1, N_DEV):
            peer = (my + d) % N_DEV
            rdma = pltpu.make_async_remote_copy(
                src_ref=amax_ref,
                dst_ref=gather.at[d - 1],
                send_sem=send_sems.at[d - 1],
                recv_sem=recv_sems.at[d - 1],
                device_id=(peer,),
                device_id_type=pl.DeviceIdType.MESH,
            )
            rdma.start()
            rdmas.append(rdma)
        for rdma in rdmas:
            rdma.wait()
        scale_ref[0, 0] = jnp.max(gather[...]) / 448.0

    s = scale_ref[0, 0]
    q = (y_ref[...] / s).astype(jnp.float8_e4m3fn)
    out_ref[...] = q.astype(jnp.float32) * s


def kernel(x, w_mat):
    a = pl.pallas_call(
        _a2a_body,
        out_shape=jax.ShapeDtypeStruct((N_DEV, M_BLK, K_BLK), jnp.bfloat16),
        in_specs=[pl.BlockSpec(memory_space=pltpu.VMEM)],
        out_specs=pl.BlockSpec(memory_space=pltpu.VMEM),
        scratch_shapes=[
            pltpu.VMEM((N_DEV, M_BLK, K_BLK), jnp.bfloat16),
            pltpu.SemaphoreType.DMA((N_DEV - 1,)),
            pltpu.SemaphoreType.DMA((N_DEV - 1,)),
        ],
        compiler_params=pltpu.CompilerParams(collective_id=0),
    )(x)

    y, amax = pl.pallas_call(
        _gemm_body,
        grid=(N_TILES,),
        out_shape=[
            jax.ShapeDtypeStruct((M_BLK, N), jnp.float32),
            jax.ShapeDtypeStruct((8, 128), jnp.float32),
        ],
        in_specs=[
            pl.BlockSpec((N_DEV, M_BLK, K_BLK), lambda n: (0, 0, 0)),
            pl.BlockSpec((K, NT), lambda n: (0, n)),
        ],
        out_specs=[
            pl.BlockSpec((M_BLK, NT), lambda n: (0, n)),
            pl.BlockSpec((8, 128), lambda n: (0, 0)),
        ],
        compiler_params=pltpu.CompilerParams(
            dimension_semantics=("arbitrary",),
            vmem_limit_bytes=60 * 1024 * 1024,
        ),
    )(a, w_mat)

    out = pl.pallas_call(
        _quant_body,
        grid=(N_TILES,),
        out_shape=jax.ShapeDtypeStruct((M_BLK, N), jnp.float32),
        in_specs=[
            pl.BlockSpec((M_BLK, NT), lambda t: (0, t)),
            pl.BlockSpec((8, 128), lambda t: (0, 0)),
        ],
        out_specs=pl.BlockSpec((M_BLK, NT), lambda t: (0, t)),
        scratch_shapes=[
            pltpu.VMEM((N_DEV, 8, 128), jnp.float32),
            pltpu.SMEM((1, 1), jnp.float32),
            pltpu.SemaphoreType.DMA((N_DEV - 1,)),
            pltpu.SemaphoreType.DMA((N_DEV - 1,)),
        ],
        compiler_params=pltpu.CompilerParams(
            dimension_semantics=("arbitrary",),
            collective_id=1,
        ),
    )(y, amax)
    return out


# device time: 189336 ns/iter; 1.0465x vs baseline; 1.0465x over previous
import jax
import jax.numpy as jnp
from jax import lax
from jax.experimental import pallas as pl
from jax.experimental.pallas import tpu as pltpu

N_DEV = 4
M_BLK = 1024
K_BLK = 1024
K = 4096
N = 8192
NT = 512
N_TILES = N // NT

OFF = (0, 1, 3, 2)
SLOT_FOR_D = {1: 2, 2: 3, 3: 1}


def _body(perm_ref, x_ref, w_ref, out_hbm,
          a_ref, acc_ref, amax_ref, gather, stage, scale_ref,
          a2a_send, a2a_recv, amax_send, amax_recv, out_sems):
    kb = pl.program_id(0)
    n = pl.program_id(1)
    my = lax.axis_index("i")

    @pl.when((kb == 0) & (n == 0))
    def _():
        barrier = pltpu.get_barrier_semaphore()
        for d in range(1, N_DEV):
            peer = (my + d) % N_DEV
            pl.semaphore_signal(
                barrier, inc=1, device_id=(peer,),
                device_id_type=pl.DeviceIdType.MESH,
            )
        pl.semaphore_wait(barrier, N_DEV - 1)

        a_ref[0] = x_ref[pl.ds(my * M_BLK, M_BLK), :]
        for d in range(1, N_DEV):
            peer = (my + d) % N_DEV
            s = SLOT_FOR_D[d]
            pltpu.make_async_remote_copy(
                src_ref=x_ref.at[pl.ds(peer * M_BLK, M_BLK), :],
                dst_ref=a_ref.at[s],
                send_sem=a2a_send.at[d - 1],
                recv_sem=a2a_recv.at[s],
                device_id=(peer,),
                device_id_type=pl.DeviceIdType.MESH,
            ).start()

    @pl.when((kb >= 1) & (kb <= 3) & (n == 0))
    def _():
        pltpu.make_async_remote_copy(
            src_ref=a_ref.at[kb],
            dst_ref=a_ref.at[kb],
            send_sem=a2a_send.at[0],
            recv_sem=a2a_recv.at[kb],
            device_id=(my,),
            device_id_type=pl.DeviceIdType.MESH,
        ).wait_recv()

    nsl = pl.ds(n * NT, NT)

    @pl.when(kb == 0)
    def _():
        acc_ref[:, nsl] = jnp.dot(
            a_ref[kb], w_ref[...].astype(jnp.bfloat16),
            preferred_element_type=jnp.float32,
        )

    @pl.when((kb >= 1) & (kb <= 3))
    def _():
        acc_ref[:, nsl] = acc_ref[:, nsl] + jnp.dot(
            a_ref[kb], w_ref[...].astype(jnp.bfloat16),
            preferred_element_type=jnp.float32,
        )

    @pl.when(kb == 3)
    def _():
        m = jnp.max(jnp.maximum(acc_ref[:, nsl], 0.0))

        @pl.when(n == 0)
        def _():
            amax_ref[...] = jnp.full((8, 128), m, jnp.float32)

        @pl.when(n > 0)
        def _():
            amax_ref[...] = jnp.maximum(amax_ref[...], m)

    @pl.when((kb == 4) & (n == 0))
    def _():
        gather[0] = amax_ref[...]
        sends = []
        for d in range(1, N_DEV):
            peer = (my + d) % N_DEV
            s = SLOT_FOR_D[d]
            rdma = pltpu.make_async_remote_copy(
                src_ref=amax_ref,
                dst_ref=gather.at[s],
                send_sem=amax_send.at[d - 1],
                recv_sem=amax_recv.at[s],
                device_id=(peer,),
                device_id_type=pl.DeviceIdType.MESH,
            )
            rdma.start()
            sends.append(rdma)
        for s in range(1, N_DEV):
            pltpu.make_async_remote_copy(
                src_ref=amax_ref,
                dst_ref=gather.at[s],
                send_sem=amax_send.at[0],
                recv_sem=amax_recv.at[s],
                device_id=(my,),
                device_id_type=pl.DeviceIdType.MESH,
            ).wait_recv()
        for rdma in sends:
            rdma.wait_send()
        for d in range(1, N_DEV):
            peer = (my + d) % N_DEV
            pltpu.make_async_remote_copy(
                src_ref=x_ref.at[pl.ds(peer * M_BLK, M_BLK), :],
                dst_ref=a_ref.at[SLOT_FOR_D[d]],
                send_sem=a2a_send.at[d - 1],
                recv_sem=a2a_recv.at[SLOT_FOR_D[d]],
                device_id=(peer,),
                device_id_type=pl.DeviceIdType.MESH,
            ).wait_send()
        scale_ref[0, 0] = jnp.max(gather[...]) / 448.0

    @pl.when(kb == 4)
    def _():
        slot = n % 2

        @pl.when(n >= 2)
        def _():
            pltpu.make_async_copy(
                stage.at[slot],
                out_hbm.at[:, pl.ds((n - 2) * NT, NT)],
                out_sems.at[slot],
            ).wait()

        s = scale_ref[0, 0]
        y = jnp.maximum(acc_ref[:, nsl], 0.0)
        q = (y * (1.0 / s)).astype(jnp.float8_e4m3fn)
        stage[slot] = q.astype(jnp.float32) * s
        pltpu.make_async_copy(
            stage.at[slot], out_hbm.at[:, nsl], out_sems.at[slot]
        ).start()

        @pl.when(n == N_TILES - 1)
        def _():
            for back in (1, 0):
                pltpu.make_async_copy(
                    stage.at[(n - back) % 2],
                    out_hbm.at[:, pl.ds((n - back) * NT, NT)],
                    out_sems.at[(n - back) % 2],
                ).wait()


def _w_index_map(kb, n, perm_ref):
    return (perm_ref[jnp.minimum(kb, 3)], jnp.where(kb == 4, 0, n))


def kernel(x, w_mat):
    my = lax.axis_index("i")
    perm = (jnp.array(OFF, dtype=jnp.int32) + my) % N_DEV
    x_bf = x.astype(jnp.bfloat16)

    return pl.pallas_call(
        _body,
        grid_spec=pltpu.PrefetchScalarGridSpec(
            num_scalar_prefetch=1,
            grid=(N_DEV + 1, N_TILES),
            in_specs=[
                pl.BlockSpec(memory_space=pltpu.VMEM),
                pl.BlockSpec((K_BLK, NT), _w_index_map),
            ],
            out_specs=pl.BlockSpec(memory_space=pl.ANY),
            scratch_shapes=[
                pltpu.VMEM((N_DEV, M_BLK, K_BLK), jnp.bfloat16),
                pltpu.VMEM((M_BLK, N), jnp.float32),
                pltpu.VMEM((8, 128), jnp.float32),
                pltpu.VMEM((N_DEV, 8, 128), jnp.float32),
                pltpu.VMEM((2, M_BLK, NT), jnp.float32),
                pltpu.SMEM((1, 1), jnp.float32),
                pltpu.SemaphoreType.DMA((N_DEV - 1,)),
                pltpu.SemaphoreType.DMA((N_DEV,)),
                pltpu.SemaphoreType.DMA((N_DEV - 1,)),
                pltpu.SemaphoreType.DMA((N_DEV,)),
                pltpu.SemaphoreType.DMA((2,)),
            ],
        ),
        out_shape=jax.ShapeDtypeStruct((M_BLK, N), jnp.float32),
        compiler_params=pltpu.CompilerParams(
            dimension_semantics=("arbitrary", "arbitrary"),
            collective_id=0,
            vmem_limit_bytes=60 * 1024 * 1024,
        ),
    )(perm, x_bf, w_mat)


# device time: 160735 ns/iter; 1.2327x vs baseline; 1.1779x over previous
import jax
import jax.numpy as jnp
from jax import lax
from jax.experimental import pallas as pl
from jax.experimental.pallas import tpu as pltpu

N_DEV = 4
M_BLK = 1024
K_BLK = 1024
K = 4096
N = 8192
NT = 512
N_TILES = N // NT

OFF = (0, 1, 3, 2)
SLOT_FOR_D = {1: 2, 2: 3, 3: 1}
SEND_ORDER = (3, 1, 2)


def _body(perm_ref, x_ref, w0_ref, w1_ref, out_hbm,
          a_ref, acc_ref, amax_ref, gather, stage, scale_ref,
          a2a_send, a2a_recv, amax_send, amax_recv, out_sems):
    kb = pl.program_id(0)
    n = pl.program_id(1)
    my = lax.axis_index("i")
    nsl = pl.ds(n * NT, NT)

    def _wait_slot(s):
        pltpu.make_async_remote_copy(
            src_ref=a_ref.at[s],
            dst_ref=a_ref.at[s],
            send_sem=a2a_send.at[0],
            recv_sem=a2a_recv.at[s],
            device_id=(my,),
            device_id_type=pl.DeviceIdType.MESH,
        ).wait_recv()

    @pl.when((kb == 0) & (n == 0))
    def _():
        barrier = pltpu.get_barrier_semaphore()
        for d in range(1, N_DEV):
            peer = (my + d) % N_DEV
            pl.semaphore_signal(
                barrier, inc=1, device_id=(peer,),
                device_id_type=pl.DeviceIdType.MESH,
            )
        pl.semaphore_wait(barrier, N_DEV - 1)

        a_ref[0] = x_ref[pl.ds(my * M_BLK, M_BLK), :]
        for d in SEND_ORDER:
            peer = (my + d) % N_DEV
            s = SLOT_FOR_D[d]
            pltpu.make_async_remote_copy(
                src_ref=x_ref.at[pl.ds(peer * M_BLK, M_BLK), :],
                dst_ref=a_ref.at[s],
                send_sem=a2a_send.at[d - 1],
                recv_sem=a2a_recv.at[s],
                device_id=(peer,),
                device_id_type=pl.DeviceIdType.MESH,
            ).start()

    @pl.when(kb == 0)
    def _():
        acc_ref[:, nsl] = jnp.dot(
            a_ref[0], w0_ref[...].astype(jnp.bfloat16),
            preferred_element_type=jnp.float32,
        )

    @pl.when((kb == 1) & (n == 0))
    def _():
        _wait_slot(1)
        _wait_slot(2)

    @pl.when(kb == 1)
    def _():
        acc_ref[:, nsl] = acc_ref[:, nsl] + jnp.dot(
            a_ref[1], w0_ref[...].astype(jnp.bfloat16),
            preferred_element_type=jnp.float32,
        ) + jnp.dot(
            a_ref[2], w1_ref[...].astype(jnp.bfloat16),
            preferred_element_type=jnp.float32,
        )

    @pl.when((kb == 2) & (n == 0))
    def _():
        _wait_slot(3)

    @pl.when(kb == 2)
    def _():
        acc_ref[:, nsl] = acc_ref[:, nsl] + jnp.dot(
            a_ref[3], w0_ref[...].astype(jnp.bfloat16),
            preferred_element_type=jnp.float32,
        )
        m = jnp.max(jnp.maximum(acc_ref[:, nsl], 0.0))

        @pl.when(n == 0)
        def _():
            amax_ref[...] = jnp.full((8, 128), m, jnp.float32)

        @pl.when(n > 0)
        def _():
            amax_ref[...] = jnp.maximum(amax_ref[...], m)

    @pl.when((kb == 3) & (n == 0))
    def _():
        gather[0] = amax_ref[...]
        sends = []
        for d in range(1, N_DEV):
            peer = (my + d) % N_DEV
            s = SLOT_FOR_D[d]
            rdma = pltpu.make_async_remote_copy(
                src_ref=amax_ref,
                dst_ref=gather.at[s],
                send_sem=amax_send.at[d - 1],
                recv_sem=amax_recv.at[s],
                device_id=(peer,),
                device_id_type=pl.DeviceIdType.MESH,
            )
            rdma.start()
            sends.append(rdma)
        for s in range(1, N_DEV):
            pltpu.make_async_remote_copy(
                src_ref=amax_ref,
                dst_ref=gather.at[s],
                send_sem=amax_send.at[0],
                recv_sem=amax_recv.at[s],
                device_id=(my,),
                device_id_type=pl.DeviceIdType.MESH,
            ).wait_recv()
        for rdma in sends:
            rdma.wait_send()
        for d in range(1, N_DEV):
            peer = (my + d) % N_DEV
            pltpu.make_async_remote_copy(
                src_ref=x_ref.at[pl.ds(peer * M_BLK, M_BLK), :],
                dst_ref=a_ref.at[SLOT_FOR_D[d]],
                send_sem=a2a_send.at[d - 1],
                recv_sem=a2a_recv.at[SLOT_FOR_D[d]],
                device_id=(peer,),
                device_id_type=pl.DeviceIdType.MESH,
            ).wait_send()
        scale_ref[0, 0] = jnp.max(gather[...]) / 448.0

    @pl.when(kb == 3)
    def _():
        slot = n % 2

        @pl.when(n >= 2)
        def _():
            pltpu.make_async_copy(
                stage.at[slot],
                out_hbm.at[:, pl.ds((n - 2) * NT, NT)],
                out_sems.at[slot],
            ).wait()

        s = scale_ref[0, 0]
        y = jnp.maximum(acc_ref[:, nsl], 0.0)
        q = (y * (1.0 / s)).astype(jnp.float8_e4m3fn)
        stage[slot] = q.astype(jnp.float32) * s
        pltpu.make_async_copy(
            stage.at[slot], out_hbm.at[:, nsl], out_sems.at[slot]
        ).start()

        @pl.when(n == N_TILES - 1)
        def _():
            for back in (1, 0):
                pltpu.make_async_copy(
                    stage.at[(n - back) % 2],
                    out_hbm.at[:, pl.ds((n - back) * NT, NT)],
                    out_sems.at[(n - back) % 2],
                ).wait()


def _w0_index_map(kb, n, perm_ref):
    k = perm_ref[jnp.where(kb == 0, 0, jnp.where(kb == 1, 1, 3))]
    return (k, jnp.where(kb == 3, 0, n))


def _w1_index_map(kb, n, perm_ref):
    return (perm_ref[2], jnp.where(kb == 1, n, 0))


def kernel(x, w_mat):
    my = lax.axis_index("i")
    perm = (jnp.array(OFF, dtype=jnp.int32) + my) % N_DEV
    x_bf = x.astype(jnp.bfloat16)

    return pl.pallas_call(
        _body,
        grid_spec=pltpu.PrefetchScalarGridSpec(
            num_scalar_prefetch=1,
            grid=(N_DEV, N_TILES),
            in_specs=[
                pl.BlockSpec(memory_space=pltpu.VMEM),
                pl.BlockSpec((K_BLK, NT), _w0_index_map),
                pl.BlockSpec((K_BLK, NT), _w1_index_map),
            ],
            out_specs=pl.BlockSpec(memory_space=pl.ANY),
            scratch_shapes=[
                pltpu.VMEM((N_DEV, M_BLK, K_BLK), jnp.bfloat16),
                pltpu.VMEM((M_BLK, N), jnp.float32),
                pltpu.VMEM((8, 128), jnp.float32),
                pltpu.VMEM((N_DEV, 8, 128), jnp.float32),
                pltpu.VMEM((2, M_BLK, NT), jnp.float32),
                pltpu.SMEM((1, 1), jnp.float32),
                pltpu.SemaphoreType.DMA((N_DEV - 1,)),
                pltpu.SemaphoreType.DMA((N_DEV,)),
                pltpu.SemaphoreType.DMA((N_DEV - 1,)),
                pltpu.SemaphoreType.DMA((N_DEV,)),
                pltpu.SemaphoreType.DMA((2,)),
            ],
        ),
        out_shape=jax.ShapeDtypeStruct((M_BLK, N), jnp.float32),
        compiler_params=pltpu.CompilerParams(
            dimension_semantics=("arbitrary", "arbitrary"),
            collective_id=0,
            vmem_limit_bytes=62 * 1024 * 1024,
        ),
    )(perm, x_bf, w_mat, w_mat)
